# baseline (speedup 1.0000x reference)
import functools

import jax
import jax.numpy as jnp
from jax import lax
from jax.experimental import pallas as pl
from jax.experimental.pallas import tpu as pltpu
from jax.experimental.pallas import tpu_sc as plsc

NC = 2
NS = 16
NW = NC * NS
C = 64
L = 16


def _pad_edges(src, dst, n_nodes, cpt):
    e = src.shape[0]
    pad = NW * cpt * C - e
    if pad:
        pad_src = jnp.arange(pad, dtype=jnp.int32) % 16
        pad_dst = n_nodes + (jnp.arange(pad, dtype=jnp.int32) % C)
        src = jnp.concatenate([src, pad_src])
        dst = jnp.concatenate([dst, pad_dst])
    srcp = src.reshape(NW, cpt, C)
    dstp = dst.reshape(NW, cpt, C)
    core_of_wid = jnp.arange(NW, dtype=jnp.int32) // NS
    srcp = srcp + core_of_wid[:, None, None] * n_nodes
    return srcp, dstp


def _fast_rsqrt(x16):
    i = lax.bitcast_convert_type(x16, jnp.int32)
    i = jnp.int32(0x5F3759DF) - lax.shift_right_arithmetic(i, 1)
    y = lax.bitcast_convert_type(i, jnp.float32)
    for _ in range(3):
        y = y * (1.5 - 0.5 * x16 * y * y)
    return y


def _zero_acc_fire(sid, acc_sh, rows3, zrows, zsem, rows_per_tile, zr):
    zero16 = jnp.zeros((L,), jnp.float32)
    d = zrows.shape[1]

    def zfill_small(i, carry):
        zrows[i // (d // L), pl.ds((i % (d // L)) * L, L)] = zero16
        return carry
    lax.fori_loop(0, zr * (d // L), zfill_small, 0)

    def zfill_big(i, carry):
        rows3[0, i // (d // L), pl.ds((i % (d // L)) * L, L)] = zero16
        return carry
    lax.fori_loop(0, C * (d // L), zfill_big, 0)

    nzf = rows_per_tile // C
    nzr = (rows_per_tile - nzf * C) // zr

    def zcopy(kk, carry):
        r0 = pl.multiple_of(sid * rows_per_tile + kk * C, 8)
        pltpu.async_copy(rows3.at[0], acc_sh.at[pl.ds(r0, C)], zsem)
        return carry
    lax.fori_loop(0, nzf, zcopy, 0)

    def zcopy2(kk, carry):
        r0 = pl.multiple_of(sid * rows_per_tile + nzf * C + kk * zr, 8)
        pltpu.async_copy(zrows, acc_sh.at[pl.ds(r0, zr)], zsem)
        return carry
    lax.fori_loop(0, nzr, zcopy2, 0)
    return nzf, nzr


def _zero_acc_drain(sid, acc_sh, rows3, zrows, zsem, rows_per_tile, zr,
                    nzf, nzr):
    def zdrain(kk, carry):
        pltpu.make_async_copy(
            rows3.at[0],
            acc_sh.at[pl.ds(pl.multiple_of(sid * rows_per_tile, 8), C)],
            zsem).wait()
        return carry
    lax.fori_loop(0, nzf, zdrain, 0)

    def zdrain2(kk, carry):
        pltpu.make_async_copy(
            zrows,
            acc_sh.at[pl.ds(pl.multiple_of(sid * rows_per_tile, 8), zr)],
            zsem).wait()
        return carry
    lax.fori_loop(0, nzr, zdrain2, 0)


def _segsum_main(h_hbm, srcb, dstb, acc_sh, rows3, gsem, ssem, cpt):
    def prime(b, carry):
        pltpu.async_copy(h_hbm.at[srcb.at[b]], rows3.at[b], gsem.at[b])
        return carry
    lax.fori_loop(0, 2, prime, 0)

    plsc.subcore_barrier()

    def body(g, carry):
        b = lax.rem(g, 3)
        bn = lax.rem(g + 2, 3)
        pltpu.make_async_copy(h_hbm.at[srcb.at[g]], rows3.at[b],
                              gsem.at[b]).wait()

        @pl.when(g >= 1)
        def _():
            pltpu.make_async_copy(rows3.at[bn], acc_sh.at[dstb.at[g - 1]],
                                  ssem.at[bn]).wait()
        pltpu.async_copy(rows3.at[b], acc_sh.at[dstb.at[g]],
                         ssem.at[b], add=True)

        @pl.when(g + 2 < cpt)
        def _():
            pltpu.async_copy(h_hbm.at[srcb.at[g + 2]], rows3.at[bn],
                             gsem.at[bn])
        return carry
    lax.fori_loop(0, cpt, body, 0)
    bl = lax.rem(cpt - 1, 3)
    pltpu.make_async_copy(rows3.at[bl], acc_sh.at[dstb.at[cpt - 1]],
                          ssem.at[bl]).wait()


def _writeback(cid, sid, acc_sh, up_hbm, wsem, n_nodes):
    wb_chunk = 128
    wb_full = n_nodes // wb_chunk
    wb_tail = n_nodes - wb_full * wb_chunk

    def wb(j, carry):
        c = j * NS + sid

        @pl.when(c < wb_full)
        def _():
            r0 = pl.multiple_of(c * wb_chunk, 8)
            pltpu.async_copy(acc_sh.at[pl.ds(r0, wb_chunk)],
                             up_hbm.at[pl.ds(cid * n_nodes + r0, wb_chunk)],
                             wsem)
        return carry
    lax.fori_loop(0, -(-wb_full // NS), wb, 0)

    if wb_tail:
        @pl.when(sid == NS - 1)
        def _():
            r0 = pl.multiple_of(wb_full * wb_chunk, 8)
            pltpu.async_copy(acc_sh.at[pl.ds(r0, wb_tail)],
                             up_hbm.at[pl.ds(cid * n_nodes + r0, wb_tail)],
                             wsem)

    def wbdrain(j, carry):
        c = j * NS + sid

        @pl.when(c < wb_full)
        def _():
            r0 = pl.multiple_of(c * wb_chunk, 8)
            pltpu.make_async_copy(
                acc_sh.at[pl.ds(r0, wb_chunk)],
                up_hbm.at[pl.ds(cid * n_nodes + r0, wb_chunk)], wsem).wait()
        return carry
    lax.fori_loop(0, -(-wb_full // NS), wbdrain, 0)

    if wb_tail:
        @pl.when(sid == NS - 1)
        def _():
            r0 = pl.multiple_of(wb_full * wb_chunk, 8)
            pltpu.make_async_copy(
                acc_sh.at[pl.ds(r0, wb_tail)],
                up_hbm.at[pl.ds(cid * n_nodes + r0, wb_tail)], wsem).wait()


def _make_kernel_a(n_nodes, n_pad, d, cpt):
    mesh = plsc.VectorSubcoreMesh(core_axis_name="c", subcore_axis_name="s")
    rows_per_tile = n_pad // NS
    zr = 8
    npre = min(4, cpt)
    nsc = n_nodes // C

    @functools.partial(
        pl.kernel,
        out_type=(
            jax.ShapeDtypeStruct((2 * n_nodes, d), jnp.float32),
            jax.ShapeDtypeStruct((2 * n_nodes, d), jnp.float32),
            jax.ShapeDtypeStruct((n_nodes,), jnp.float32),
        ),
        mesh=mesh,
        scratch_types=[
            pltpu.VMEM_SHARED((n_pad, d), jnp.float32),
            pltpu.VMEM_SHARED((n_pad,), jnp.float32),
            pltpu.VMEM((cpt, C), jnp.int32),
            pltpu.VMEM((cpt, C), jnp.int32),
            pltpu.VMEM((3, C, d), jnp.float32),
            pltpu.VMEM((zr, d), jnp.float32),
            pltpu.VMEM((2048,), jnp.float32),
            pltpu.VMEM((C,), jnp.float32),
            pltpu.VMEM((C,), jnp.float32),
            pltpu.VMEM((C,), jnp.float32),
            pltpu.SemaphoreType.DMA((3,)),
            pltpu.SemaphoreType.DMA((3,)),
            pltpu.SemaphoreType.DMA,
            pltpu.SemaphoreType.DMA,
        ],
        compiler_params=pltpu.CompilerParams(use_tc_tiling_on_sc=False),
    )
    def kernel_a(x_hbm, srcp_hbm, dstp_hbm, u1p_hbm, a0p_hbm, dinv_hbm,
                 acc_sh, deg_sh, srcb, dstb, rows3, zrows, zvd, onesv,
                 degv, dinvb, gsem, ssem, zsem, dsem):
        cid = lax.axis_index("c")
        sid = lax.axis_index("s")
        wid = cid * NS + sid
        wid_other = lax.rem(wid + NS, NW)

        ones16 = jnp.ones((L,), jnp.float32)
        zero16 = jnp.zeros((L,), jnp.float32)
        for j in range(C // L):
            onesv[pl.ds(j * L, L)] = ones16

        def zvdfill(i, carry):
            zvd[pl.ds(i * L, L)] = zero16
            return carry
        lax.fori_loop(0, 2048 // L, zvdfill, 0)

        pltpu.sync_copy(srcp_hbm.at[wid], srcb)
        pltpu.sync_copy(dstp_hbm.at[wid_other], dstb)

        @pl.when(sid < 4)
        def _():
            pltpu.sync_copy(zvd, deg_sh.at[pl.ds(sid * 2048, 2048)])

        @pl.when(sid == 4)
        def _():
            pltpu.sync_copy(zvd.at[pl.ds(0, n_pad - 4 * 2048)],
                            deg_sh.at[pl.ds(4 * 2048, n_pad - 4 * 2048)])

        nzf, nzr = _zero_acc_fire(sid, acc_sh, rows3, zrows, zsem,
                                  rows_per_tile, zr)

        plsc.subcore_barrier()

        def deg_half():
            for b in range(npre):
                pltpu.async_copy(onesv, deg_sh.at[dstb.at[b]], dsem,
                                 add=True)

            def dbody(g, carry):
                pltpu.make_async_copy(onesv, deg_sh.at[dstb.at[g]],
                                      dsem).wait()
                pltpu.async_copy(onesv, deg_sh.at[dstb.at[g + npre]], dsem,
                                 add=True)
                return carry
            lax.fori_loop(0, cpt - npre, dbody, 0)

            def ddrain(g, carry):
                pltpu.make_async_copy(onesv, deg_sh.at[dstb.at[0]],
                                      dsem).wait()
                return carry
            lax.fori_loop(0, npre, ddrain, 0)

        deg_half()
        pltpu.sync_copy(dstp_hbm.at[wid], dstb)
        deg_half()

        plsc.subcore_barrier()

        def scale_chunk(c, carry):
            r0 = pl.multiple_of(
                jnp.where(c < nsc, c * C, n_nodes - C).astype(jnp.int32), 8)
            pltpu.sync_copy(deg_sh.at[pl.ds(r0, C)], degv)
            pltpu.sync_copy(x_hbm.at[pl.ds(r0, C)], rows3.at[1])
            for j in range(C // L):
                dv = degv[pl.ds(j * L, L)]
                dinvb[pl.ds(j * L, L)] = _fast_rsqrt(jnp.maximum(dv, 1.0))

            def srow_grp(q, carry2):
                sv16 = dinvb[pl.ds(q * L, L)]
                for i in range(L):
                    r = q * L + i
                    sv = jnp.full((L,), sv16[i], jnp.float32)
                    for j in range(d // L):
                        rows3[1, r, pl.ds(j * L, L)] = (
                            rows3[1, r, pl.ds(j * L, L)] * sv)
                return carry2
            lax.fori_loop(0, C // L, srow_grp, 0)
            pltpu.sync_copy(rows3.at[1],
                            a0p_hbm.at[pl.ds(cid * n_nodes + r0, C)])

            @pl.when(cid == 0)
            def _():
                pltpu.sync_copy(dinvb, dinv_hbm.at[pl.ds(r0, C)])
            return carry

        def scale_loop(j, carry):
            c = j * NS + sid

            @pl.when(c <= nsc)
            def _():
                scale_chunk(c, 0)
            return carry
        lax.fori_loop(0, -(-(nsc + 1) // NS), scale_loop, 0)

        _zero_acc_drain(sid, acc_sh, rows3, zrows, zsem, rows_per_tile, zr,
                        nzf, nzr)
        plsc.subcore_barrier()

        _segsum_main(a0p_hbm, srcb, dstb, acc_sh, rows3, gsem, ssem, cpt)

        plsc.subcore_barrier()
        _writeback(cid, sid, acc_sh, u1p_hbm, zsem, n_nodes)

    return kernel_a


def _make_kernel_b(n_nodes, n_pad, d, cpt):
    mesh = plsc.VectorSubcoreMesh(core_axis_name="c", subcore_axis_name="s")
    rows_per_tile = n_pad // NS
    zr = 8
    nsc = n_nodes // C

    @functools.partial(
        pl.kernel,
        out_type=(
            jax.ShapeDtypeStruct((2 * n_nodes, d), jnp.float32),
            jax.ShapeDtypeStruct((2 * n_nodes, d), jnp.float32),
        ),
        mesh=mesh,
        scratch_types=[
            pltpu.VMEM_SHARED((n_pad, d), jnp.float32),
            pltpu.VMEM((cpt, C), jnp.int32),
            pltpu.VMEM((cpt, C), jnp.int32),
            pltpu.VMEM((3, C, d), jnp.float32),
            pltpu.VMEM((zr, d), jnp.float32),
            pltpu.VMEM((C,), jnp.float32),
            pltpu.SemaphoreType.DMA((3,)),
            pltpu.SemaphoreType.DMA((3,)),
            pltpu.SemaphoreType.DMA,
        ],
        compiler_params=pltpu.CompilerParams(use_tc_tiling_on_sc=False),
    )
    def kernel_b(u1p_hbm, dinv_hbm, srcp_hbm, dstp_hbm, u2p_hbm, a1p_hbm,
                 acc_sh, srcb, dstb, rows3, zrows, dinvb, gsem, ssem, zsem):
        cid = lax.axis_index("c")
        sid = lax.axis_index("s")
        wid = cid * NS + sid

        pltpu.sync_copy(srcp_hbm.at[wid], srcb)
        pltpu.sync_copy(dstp_hbm.at[wid], dstb)

        nzf, nzr = _zero_acc_fire(sid, acc_sh, rows3, zrows, zsem,
                                  rows_per_tile, zr)

        def scale_chunk(c, carry):
            r0 = pl.multiple_of(
                jnp.where(c < nsc, c * C, n_nodes - C).astype(jnp.int32), 8)
            pltpu.sync_copy(dinv_hbm.at[pl.ds(r0, C)], dinvb)
            pltpu.sync_copy(u1p_hbm.at[pl.ds(r0, C)], rows3.at[1])
            pltpu.sync_copy(u1p_hbm.at[pl.ds(n_nodes + r0, C)], rows3.at[2])

            def srow_grp(q, carry2):
                sv16 = dinvb[pl.ds(q * L, L)]
                nsq16 = -(sv16 * sv16)
                for i in range(L):
                    r = q * L + i
                    sv = jnp.full((L,), nsq16[i], jnp.float32)
                    for j in range(d // L):
                        rows3[1, r, pl.ds(j * L, L)] = (
                            (rows3[1, r, pl.ds(j * L, L)]
                             + rows3[2, r, pl.ds(j * L, L)]) * sv)
                return carry2
            lax.fori_loop(0, C // L, srow_grp, 0)
            pltpu.sync_copy(rows3.at[1],
                            a1p_hbm.at[pl.ds(cid * n_nodes + r0, C)])
            return carry

        def scale_loop(j, carry):
            c = j * NS + sid

            @pl.when(c <= nsc)
            def _():
                scale_chunk(c, 0)
            return carry
        lax.fori_loop(0, -(-(nsc + 1) // NS), scale_loop, 0)

        _zero_acc_drain(sid, acc_sh, rows3, zrows, zsem, rows_per_tile, zr,
                        nzf, nzr)
        plsc.subcore_barrier()

        _segsum_main(a1p_hbm, srcb, dstb, acc_sh, rows3, gsem, ssem, cpt)

        plsc.subcore_barrier()
        _writeback(cid, sid, acc_sh, u2p_hbm, zsem, n_nodes)

    return kernel_b


def _out_body(x_ref, u1p_ref, u2p_ref, dinv_ref, fc_ref, w_ref, b_ref,
              out_ref):
    dinv = dinv_ref[...]
    tx1 = -(dinv * (u1p_ref[0] + u1p_ref[1]))
    tx2 = -2.0 * (dinv * (u2p_ref[0] + u2p_ref[1])) - x_ref[...]
    acc = jnp.dot(fc_ref[0] * x_ref[...], w_ref[0],
                  preferred_element_type=jnp.float32)
    acc = acc + jnp.dot(fc_ref[1] * tx1, w_ref[1],
                        preferred_element_type=jnp.float32)
    acc = acc + jnp.dot(fc_ref[2] * tx2, w_ref[2],
                        preferred_element_type=jnp.float32)
    out_ref[...] = acc + b_ref[...]


def kernel(x, edge_index, filter_coeff, weight, bias):
    n, d = x.shape
    e = edge_index.shape[1]
    k = weight.shape[0]
    assert (n, d, k) == (10000, 128, 3), "kernel specialized to fixed shapes"

    src = edge_index[0].astype(jnp.int32)
    dst = edge_index[1].astype(jnp.int32)

    cpt = -(-e // (NW * C))
    n_pad = 10112
    srcp, dstp = _pad_edges(src, dst, n, cpt)

    u1p, _a0p, dinv = _make_kernel_a(n, n_pad, d, cpt)(x, srcp, dstp)
    u2p, _a1p = _make_kernel_b(n, n_pad, d, cpt)(u1p, dinv, srcp, dstp)

    br = 2000
    grid = (n // br,)
    fc3 = filter_coeff.reshape(k, n, 1)
    bias2 = bias.reshape(1, d)
    out = pl.pallas_call(
        _out_body,
        grid=grid,
        in_specs=[
            pl.BlockSpec((br, d), lambda i: (i, 0)),
            pl.BlockSpec((2, br, d), lambda i: (0, i, 0)),
            pl.BlockSpec((2, br, d), lambda i: (0, i, 0)),
            pl.BlockSpec((br, 1), lambda i: (i, 0)),
            pl.BlockSpec((k, br, 1), lambda i: (0, i, 0)),
            pl.BlockSpec((k, d, d), lambda i: (0, 0, 0)),
            pl.BlockSpec((1, d), lambda i: (0, 0)),
        ],
        out_specs=pl.BlockSpec((br, d), lambda i: (i, 0)),
        out_shape=jax.ShapeDtypeStruct((n, d), jnp.float32),
    )(x, u1p.reshape(2, n, d), u2p.reshape(2, n, d), dinv.reshape(n, 1),
      fc3, weight, bias2)

    return out


# speedup vs baseline: 1.1304x; 1.1304x over previous
import functools

import jax
import jax.numpy as jnp
from jax import lax
from jax.experimental import pallas as pl
from jax.experimental.pallas import tpu as pltpu
from jax.experimental.pallas import tpu_sc as plsc

NC = 2
NS = 16
NW = NC * NS
C = 64
L = 16


def _pad_edges(src, dst, n_nodes, cpt):
    e = src.shape[0]
    pad = NW * cpt * C - e
    if pad:
        pad_src = jnp.arange(pad, dtype=jnp.int32) % 16
        pad_dst = n_nodes + (jnp.arange(pad, dtype=jnp.int32) % C)
        src = jnp.concatenate([src, pad_src])
        dst = jnp.concatenate([dst, pad_dst])
    return src.reshape(NW, cpt, C), dst.reshape(NW, cpt, C)


def _make_deg_kernel(n_nodes, n_pad, cpt):
    mesh = plsc.VectorSubcoreMesh(core_axis_name="c", subcore_axis_name="s")
    npre = min(8, cpt)

    @functools.partial(
        pl.kernel,
        out_type=jax.ShapeDtypeStruct((2 * n_nodes,), jnp.float32),
        mesh=mesh,
        scratch_types=[
            pltpu.VMEM_SHARED((n_pad,), jnp.float32),
            pltpu.VMEM((cpt, C), jnp.int32),
            pltpu.VMEM((C,), jnp.float32),
            pltpu.VMEM((2000,), jnp.float32),
            pltpu.SemaphoreType.DMA,
        ],
    )
    def deg_kernel(dstp_hbm, degp_hbm, deg_sh, idxb, onesv, zv, sem):
        cid = lax.axis_index("c")
        sid = lax.axis_index("s")
        wid = cid * NS + sid

        ones16 = jnp.ones((L,), jnp.float32)
        zero16 = jnp.zeros((L,), jnp.float32)
        for j in range(C // L):
            onesv[pl.ds(j * L, L)] = ones16

        def zfill(i, carry):
            zv[pl.ds(i * L, L)] = zero16
            return carry
        lax.fori_loop(0, 2000 // L, zfill, 0)

        @pl.when(sid < 5)
        def _():
            pltpu.sync_copy(zv, deg_sh.at[pl.ds(sid * 2000, 2000)])

        @pl.when(sid == 5)
        def _():
            pltpu.sync_copy(zv.at[pl.ds(0, n_pad - 10000)],
                            deg_sh.at[pl.ds(10000, n_pad - 10000)])

        pltpu.sync_copy(dstp_hbm.at[wid], idxb)
        plsc.subcore_barrier()

        for b in range(npre):
            pltpu.async_copy(onesv, deg_sh.at[idxb.at[b]], sem, add=True)

        def body(g, carry):
            pltpu.make_async_copy(onesv, deg_sh.at[idxb.at[g]], sem).wait()
            pltpu.async_copy(onesv, deg_sh.at[idxb.at[g + npre]], sem,
                             add=True)
            return carry
        lax.fori_loop(0, cpt - npre, body, 0)
        for b in range(npre):
            pltpu.make_async_copy(onesv, deg_sh.at[idxb.at[0]], sem).wait()

        plsc.subcore_barrier()

        @pl.when(sid < 5)
        def _():
            pltpu.sync_copy(deg_sh.at[pl.ds(sid * 2000, 2000)], zv)
            off = pl.multiple_of(cid * n_nodes + sid * 2000, 8)
            pltpu.sync_copy(zv, degp_hbm.at[pl.ds(off, 2000)])

    return deg_kernel


def _make_segsum_kernel(n_nodes, n_pad, d, cpt):
    mesh = plsc.VectorSubcoreMesh(core_axis_name="c", subcore_axis_name="s")
    rows_per_tile = n_pad // NS
    zr = 8
    wb_chunk = 128
    wb_full = n_nodes // wb_chunk
    wb_tail = n_nodes - wb_full * wb_chunk

    @functools.partial(
        pl.kernel,
        out_type=jax.ShapeDtypeStruct((2 * n_nodes, d), jnp.float32),
        mesh=mesh,
        scratch_types=[
            pltpu.VMEM_SHARED((n_pad, d), jnp.float32),
            pltpu.VMEM((cpt, C), jnp.int32),
            pltpu.VMEM((cpt, C), jnp.int32),
            pltpu.VMEM((3, C, d), jnp.float32),
            pltpu.VMEM((zr, d), jnp.float32),
            pltpu.SemaphoreType.DMA((3,)),
            pltpu.SemaphoreType.DMA((3,)),
        ],
        compiler_params=pltpu.CompilerParams(use_tc_tiling_on_sc=False),
    )
    def segsum_kernel(h_hbm, srcp_hbm, dstp_hbm, up_hbm,
                      acc_sh, srcb, dstb, rows3, zrows, gsem, ssem):
        cid = lax.axis_index("c")
        sid = lax.axis_index("s")
        wid = cid * NS + sid

        pltpu.sync_copy(srcp_hbm.at[wid], srcb)
        pltpu.sync_copy(dstp_hbm.at[wid], dstb)

        zero16 = jnp.zeros((L,), jnp.float32)

        def zfill_small(i, carry):
            zrows[i // (d // L), pl.ds((i % (d // L)) * L, L)] = zero16
            return carry
        lax.fori_loop(0, zr * (d // L), zfill_small, 0)

        def zfill_big(i, carry):
            rows3[0, i // (d // L), pl.ds((i % (d // L)) * L, L)] = zero16
            return carry
        lax.fori_loop(0, C * (d // L), zfill_big, 0)

        nzf = rows_per_tile // C
        nzr = (rows_per_tile - nzf * C) // zr

        def zcopy(kk, carry):
            r0 = pl.multiple_of(sid * rows_per_tile + kk * C, 8)
            pltpu.async_copy(rows3.at[0], acc_sh.at[pl.ds(r0, C)], ssem.at[0])
            return carry
        lax.fori_loop(0, nzf, zcopy, 0)

        def zcopy2(kk, carry):
            r0 = pl.multiple_of(
                sid * rows_per_tile + nzf * C + kk * zr, 8)
            pltpu.async_copy(zrows, acc_sh.at[pl.ds(r0, zr)], ssem.at[0])
            return carry
        lax.fori_loop(0, nzr, zcopy2, 0)

        def zdrain(kk, carry):
            pltpu.make_async_copy(
                rows3.at[0],
                acc_sh.at[pl.ds(pl.multiple_of(sid * rows_per_tile, 8), C)],
                ssem.at[0]).wait()
            return carry
        lax.fori_loop(0, nzf, zdrain, 0)

        def zdrain2(kk, carry):
            pltpu.make_async_copy(
                zrows,
                acc_sh.at[pl.ds(pl.multiple_of(sid * rows_per_tile, 8), zr)],
                ssem.at[0]).wait()
            return carry
        lax.fori_loop(0, nzr, zdrain2, 0)

        def prime(b, carry):
            pltpu.async_copy(h_hbm.at[srcb.at[b]], rows3.at[b], gsem.at[b])
            return carry
        lax.fori_loop(0, 2, prime, 0)

        plsc.subcore_barrier()

        def body(g, carry):
            b = lax.rem(g, 3)
            bn = lax.rem(g + 2, 3)
            pltpu.make_async_copy(h_hbm.at[srcb.at[g]], rows3.at[b],
                                  gsem.at[b]).wait()
            pltpu.async_copy(rows3.at[b], acc_sh.at[dstb.at[g]],
                             ssem.at[b], add=True)

            @pl.when(g >= 1)
            def _():
                pltpu.make_async_copy(rows3.at[bn],
                                      acc_sh.at[dstb.at[g - 1]],
                                      ssem.at[bn]).wait()

            @pl.when(g + 2 < cpt)
            def _():
                pltpu.async_copy(h_hbm.at[srcb.at[g + 2]], rows3.at[bn],
                                 gsem.at[bn])
            return carry
        lax.fori_loop(0, cpt, body, 0)
        bl = lax.rem(cpt - 1, 3)
        pltpu.make_async_copy(rows3.at[bl], acc_sh.at[dstb.at[cpt - 1]],
                              ssem.at[bl]).wait()

        plsc.subcore_barrier()

        def wb(j, carry):
            c = j * NS + sid

            @pl.when(c < wb_full)
            def _():
                r0 = pl.multiple_of(c * wb_chunk, 8)
                pltpu.async_copy(
                    acc_sh.at[pl.ds(r0, wb_chunk)],
                    up_hbm.at[pl.ds(cid * n_nodes + r0, wb_chunk)],
                    ssem.at[1])
            return carry
        lax.fori_loop(0, -(-wb_full // NS), wb, 0)

        if wb_tail:
            @pl.when(sid == NS - 1)
            def _():
                r0 = pl.multiple_of(wb_full * wb_chunk, 8)
                pltpu.async_copy(
                    acc_sh.at[pl.ds(r0, wb_tail)],
                    up_hbm.at[pl.ds(cid * n_nodes + r0, wb_tail)],
                    ssem.at[1])

        def wbdrain(j, carry):
            c = j * NS + sid

            @pl.when(c < wb_full)
            def _():
                r0 = pl.multiple_of(c * wb_chunk, 8)
                pltpu.make_async_copy(
                    acc_sh.at[pl.ds(r0, wb_chunk)],
                    up_hbm.at[pl.ds(cid * n_nodes + r0, wb_chunk)],
                    ssem.at[1]).wait()
            return carry
        lax.fori_loop(0, -(-wb_full // NS), wbdrain, 0)

        if wb_tail:
            @pl.when(sid == NS - 1)
            def _():
                r0 = pl.multiple_of(wb_full * wb_chunk, 8)
                pltpu.make_async_copy(
                    acc_sh.at[pl.ds(r0, wb_tail)],
                    up_hbm.at[pl.ds(cid * n_nodes + r0, wb_tail)],
                    ssem.at[1]).wait()

    return segsum_kernel


def _prep_body(degp_ref, x_ref, a0_ref, dinv_ref):
    deg = degp_ref[0] + degp_ref[1]
    dinv = lax.rsqrt(jnp.maximum(deg, 1.0))
    dinv_ref[...] = dinv
    a0_ref[...] = x_ref[...] * dinv


def _mid_body(u1p_ref, dinv_ref, tx1_ref, a1_ref):
    u = u1p_ref[0] + u1p_ref[1]
    dinv = dinv_ref[...]
    tx1 = -(dinv * u)
    tx1_ref[...] = tx1
    a1_ref[...] = dinv * tx1


def _out_body(x_ref, tx1_ref, u2p_ref, dinv_ref, fc_ref, w_ref, b_ref,
              out_ref):
    u2 = u2p_ref[0] + u2p_ref[1]
    tx2 = -2.0 * (dinv_ref[...] * u2) - x_ref[...]
    acc = jnp.dot(fc_ref[0] * x_ref[...], w_ref[0],
                  preferred_element_type=jnp.float32)
    acc = acc + jnp.dot(fc_ref[1] * tx1_ref[...], w_ref[1],
                        preferred_element_type=jnp.float32)
    acc = acc + jnp.dot(fc_ref[2] * tx2, w_ref[2],
                        preferred_element_type=jnp.float32)
    out_ref[...] = acc + b_ref[...]


def kernel(x, edge_index, filter_coeff, weight, bias):
    n, d = x.shape
    e = edge_index.shape[1]
    k = weight.shape[0]
    assert (n, d, k) == (10000, 128, 3), "kernel specialized to fixed shapes"

    src = edge_index[0].astype(jnp.int32)
    dst = edge_index[1].astype(jnp.int32)

    cpt = -(-e // (NW * C))
    n_pad = 10112
    srcp, dstp = _pad_edges(src, dst, n, cpt)

    br = 2000
    grid = (n // br,)

    degp = _make_deg_kernel(n, n_pad, cpt)(dstp)
    degp3 = degp.reshape(2, n, 1)

    a0, dinv = pl.pallas_call(
        _prep_body,
        grid=grid,
        in_specs=[
            pl.BlockSpec((2, br, 1), lambda i: (0, i, 0)),
            pl.BlockSpec((br, d), lambda i: (i, 0)),
        ],
        out_specs=[
            pl.BlockSpec((br, d), lambda i: (i, 0)),
            pl.BlockSpec((br, 1), lambda i: (i, 0)),
        ],
        out_shape=[
            jax.ShapeDtypeStruct((n, d), jnp.float32),
            jax.ShapeDtypeStruct((n, 1), jnp.float32),
        ],
    )(degp3, x)

    segsum = _make_segsum_kernel(n, n_pad, d, cpt)

    u1p = segsum(a0, srcp, dstp).reshape(2, n, d)

    tx1, a1 = pl.pallas_call(
        _mid_body,
        grid=grid,
        in_specs=[
            pl.BlockSpec((2, br, d), lambda i: (0, i, 0)),
            pl.BlockSpec((br, 1), lambda i: (i, 0)),
        ],
        out_specs=[
            pl.BlockSpec((br, d), lambda i: (i, 0)),
            pl.BlockSpec((br, d), lambda i: (i, 0)),
        ],
        out_shape=[
            jax.ShapeDtypeStruct((n, d), jnp.float32),
            jax.ShapeDtypeStruct((n, d), jnp.float32),
        ],
    )(u1p, dinv)

    u2p = segsum(a1, srcp, dstp).reshape(2, n, d)

    fc3 = filter_coeff.reshape(k, n, 1)
    bias2 = bias.reshape(1, d)
    out = pl.pallas_call(
        _out_body,
        grid=grid,
        in_specs=[
            pl.BlockSpec((br, d), lambda i: (i, 0)),
            pl.BlockSpec((br, d), lambda i: (i, 0)),
            pl.BlockSpec((2, br, d), lambda i: (0, i, 0)),
            pl.BlockSpec((br, 1), lambda i: (i, 0)),
            pl.BlockSpec((k, br, 1), lambda i: (0, i, 0)),
            pl.BlockSpec((k, d, d), lambda i: (0, 0, 0)),
            pl.BlockSpec((1, d), lambda i: (0, 0)),
        ],
        out_specs=pl.BlockSpec((br, d), lambda i: (i, 0)),
        out_shape=jax.ShapeDtypeStruct((n, d), jnp.float32),
    )(x, tx1, u2p, dinv, fc3, weight, bias2)

    return out
